# baseline (speedup 1.0000x reference)
import functools

import jax
import jax.numpy as jnp
from jax.experimental import pallas as pl
from jax.experimental.pallas import tpu as pltpu

V = 2000
NUM_ITEMS = V - 2
PAD_ID = NUM_ITEMS
HID = 32


def _loss_body(src_idx_ref, dst_idx_ref, lab_ref,
               src_row_ref, dst_row_ref, pop_ref, popn_ref,
               w1t_ref, b1_ref, w2_ref, b2_ref, out_ref):
    i = pl.program_id(0)

    pop = pop_ref[...]
    inv_pop = jnp.where(pop == 0.0, 0.0, 1.0 / jnp.where(pop == 0.0, 1.0, pop))

    a = src_row_ref[0] * inv_pop
    b = dst_row_ref[0] * inv_pop
    c = popn_ref[...]

    u = w1t_ref[:, 0:1]
    w = w1t_ref[:, 1:2]
    t = w1t_ref[:, 2:3]
    b1 = b1_ref[...]

    pre = u * a + w * b + (t * c + b1)
    g = jax.nn.gelu(pre)
    logits = jnp.sum(g * w2_ref[...], axis=0, keepdims=True) + b2_ref[...]

    col = jax.lax.broadcasted_iota(jnp.int32, (1, V), 1)
    valid = col < NUM_ITEMS
    m = jnp.max(jnp.where(valid, logits, -jnp.inf))
    s = jnp.sum(jnp.where(valid, jnp.exp(logits - m), 0.0))
    lab = lab_ref[i]
    lab_logit = jnp.sum(jnp.where(col == lab, logits, 0.0))
    loss = jnp.log(s) + m - lab_logit
    out_ref[...] = jnp.full((1, 1, 1), loss, dtype=jnp.float32)


@jax.jit
def kernel(st_src, st_dst, pop_biases, pop_biases_norm, W1, b1, W2, b2,
           masked_sequences, labels, positions):
    B, L = masked_sequences.shape
    n = B * L

    seqs = jnp.maximum(masked_sequences, 0)
    pad_col = jnp.full((B, 1), PAD_ID, dtype=seqs.dtype)
    src_idx = jnp.concatenate([pad_col, seqs[:, :-1]], axis=1).reshape(n)
    dst_idx = jnp.concatenate([seqs[:, 1:], pad_col], axis=1).reshape(n)
    lab = jnp.maximum(labels, 0).reshape(n)

    grid_spec = pltpu.PrefetchScalarGridSpec(
        num_scalar_prefetch=3,
        grid=(n,),
        in_specs=[
            pl.BlockSpec((1, 1, V), lambda i, s, d, l: (s[i], 0, 0)),
            pl.BlockSpec((1, 1, V), lambda i, s, d, l: (d[i], 0, 0)),
            pl.BlockSpec((1, V), lambda i, s, d, l: (0, 0)),
            pl.BlockSpec((1, V), lambda i, s, d, l: (0, 0)),
            pl.BlockSpec((HID, 3), lambda i, s, d, l: (0, 0)),
            pl.BlockSpec((HID, 1), lambda i, s, d, l: (0, 0)),
            pl.BlockSpec((HID, 1), lambda i, s, d, l: (0, 0)),
            pl.BlockSpec((1, 1), lambda i, s, d, l: (0, 0)),
        ],
        out_specs=pl.BlockSpec((1, 1, 1), lambda i, s, d, l: (i, 0, 0)),
    )

    out = pl.pallas_call(
        _loss_body,
        grid_spec=grid_spec,
        out_shape=jax.ShapeDtypeStruct((n, 1, 1), jnp.float32),
    )(src_idx, dst_idx, lab,
      st_src.reshape(V, 1, V), st_dst.reshape(V, 1, V), pop_biases,
      pop_biases_norm.reshape(1, V),
      W1.T, b1.reshape(HID, 1), W2, b2.reshape(1, 1))

    return out.reshape(n)


# speedup vs baseline: 2.4479x; 2.4479x over previous
import functools

import jax
import jax.numpy as jnp
from jax import lax
from jax.experimental import pallas as pl
from jax.experimental.pallas import tpu as pltpu
from jax.experimental.pallas import tpu_sc as plsc

V = 2000
V2 = 2048
NUM_ITEMS = V - 2
PAD_ID = NUM_ITEMS
HID = 32
P = 8
NPAD = 1024


def _make_sc_gather():
    info = plsc.get_sparse_core_info()
    nc, ns = info.num_cores, info.num_subcores
    nw = nc * ns
    b_per_w = NPAD // nw
    mesh = plsc.VectorSubcoreMesh(core_axis_name="c", subcore_axis_name="s")

    @functools.partial(
        pl.kernel, mesh=mesh,
        out_type=(pltpu.HBM((NPAD, V2), jnp.float32),
                  pltpu.HBM((NPAD, V2), jnp.float32)),
        scratch_types=[
            pltpu.VMEM((b_per_w,), jnp.int32),
            pltpu.VMEM((b_per_w, V2), jnp.float32),
            pltpu.SemaphoreType.DMA,
        ],
    )
    def sc_gather(src_hbm, dst_hbm, sidx_hbm, didx_hbm,
                  out_src, out_dst, idx_v, rows, sem):
        wid = lax.axis_index("s") * nc + lax.axis_index("c")
        base = wid * b_per_w
        for idx_hbm, out in ((sidx_hbm, out_src), (didx_hbm, out_dst)):
            pltpu.sync_copy(idx_hbm.at[pl.ds(base, b_per_w)], idx_v)
            pltpu.async_copy(
                src_hbm.at[idx_v] if out is out_src else dst_hbm.at[idx_v],
                rows, sem).wait()
            pltpu.sync_copy(rows, out.at[pl.ds(base, b_per_w)])

    return sc_gather


def _tc_body(src_ref, dst_ref, pop_ref, popn_ref,
             w1t_ref, b1_ref, w2_ref, b2_ref, lab_ref, out_ref):
    pop = pop_ref[...]
    inv = jnp.where(pop == 0.0, 0.0,
                    1.0 / jnp.where(pop == 0.0, 1.0, pop))
    a = src_ref[...] * inv
    b = dst_ref[...] * inv
    c8 = jnp.broadcast_to(popn_ref[...], (P, V2))

    acc = jnp.zeros((P, V2), jnp.float32)
    for h in range(HID):
        u = w1t_ref[h:h + 1, 0:1]
        w = w1t_ref[h:h + 1, 1:2]
        t = w1t_ref[h:h + 1, 2:3]
        pre = a * u + b * w + (c8 * t + b1_ref[h:h + 1, 0:1])
        acc = acc + jax.nn.gelu(pre) * w2_ref[h:h + 1, 0:1]
    logits = acc + b2_ref[0:1, 0:1]

    col = lax.broadcasted_iota(jnp.int32, (1, V2), 1)
    valid = col < NUM_ITEMS
    neg = jnp.where(valid, logits, -jnp.inf)
    m = jnp.max(neg, axis=1, keepdims=True)
    s = jnp.sum(jnp.where(valid, jnp.exp(logits - m), 0.0),
                axis=1, keepdims=True)
    lab = lab_ref[0]
    pick = jnp.sum(jnp.where(col == lab, logits, 0.0),
                   axis=1, keepdims=True)
    loss = jnp.log(s) + m - pick
    out_ref[...] = loss.reshape(1, P, 1)


def _tc_loss(src_g, dst_g, pop2, popn2, W1, b1, W2, b2, lab, n):
    nblk = n // P
    return pl.pallas_call(
        _tc_body,
        grid=(nblk,),
        in_specs=[
            pl.BlockSpec((P, V2), lambda i: (i, 0)),
            pl.BlockSpec((P, V2), lambda i: (i, 0)),
            pl.BlockSpec((1, V2), lambda i: (0, 0)),
            pl.BlockSpec((1, V2), lambda i: (0, 0)),
            pl.BlockSpec((HID, 3), lambda i: (0, 0)),
            pl.BlockSpec((HID, 1), lambda i: (0, 0)),
            pl.BlockSpec((HID, 1), lambda i: (0, 0)),
            pl.BlockSpec((1, 1), lambda i: (0, 0)),
            pl.BlockSpec((1, P, 1), lambda i: (i, 0, 0)),
        ],
        out_specs=pl.BlockSpec((1, P, 1), lambda i: (i, 0, 0)),
        out_shape=jax.ShapeDtypeStruct((nblk, P, 1), jnp.float32),
    )(src_g, dst_g, pop2, popn2,
      W1.T, b1.reshape(HID, 1), W2, b2.reshape(1, 1),
      lab.reshape(nblk, P, 1)).reshape(n)


@jax.jit
def kernel(st_src, st_dst, pop_biases, pop_biases_norm, W1, b1, W2, b2,
           masked_sequences, labels, positions):
    B, L = masked_sequences.shape
    n = B * L

    seqs = jnp.maximum(masked_sequences, 0)
    pad_col = jnp.full((B, 1), PAD_ID, dtype=seqs.dtype)
    src_idx = jnp.concatenate([pad_col, seqs[:, :-1]], axis=1).reshape(n)
    dst_idx = jnp.concatenate([seqs[:, 1:], pad_col], axis=1).reshape(n)
    zpad = jnp.zeros((NPAD - n,), jnp.int32)
    src_idx = jnp.concatenate([src_idx, zpad])
    dst_idx = jnp.concatenate([dst_idx, zpad])
    lab = jnp.maximum(labels, 0).reshape(n)

    cpad = ((0, 0), (0, V2 - V))
    src2 = jnp.pad(st_src, cpad)
    dst2 = jnp.pad(st_dst, cpad)
    pop2 = jnp.pad(pop_biases, cpad)
    popn2 = jnp.pad(pop_biases_norm.reshape(1, V), cpad)

    src_g, dst_g = _make_sc_gather()(src2, dst2, src_idx, dst_idx)

    return _tc_loss(src_g, dst_g, pop2, popn2, W1, b1, W2, b2, lab, n)


# speedup vs baseline: 3.4924x; 1.4267x over previous
import functools

import jax
import jax.numpy as jnp
from jax import lax
from jax.experimental import pallas as pl
from jax.experimental.pallas import tpu as pltpu
from jax.experimental.pallas import tpu_sc as plsc

V = 2000
V2 = 2048
NUM_ITEMS = V - 2
PAD_ID = NUM_ITEMS
HID = 32
P = 16
NPAD = 1024


def _make_sc_gather(n):
    info = plsc.get_sparse_core_info()
    nc, ns = info.num_cores, info.num_subcores
    nw = nc * ns
    b_per_w = NPAD // nw
    mesh = plsc.VectorSubcoreMesh(core_axis_name="c", subcore_axis_name="s")

    @functools.partial(
        pl.kernel, mesh=mesh,
        out_type=(pltpu.HBM((NPAD, V2), jnp.float32),
                  pltpu.HBM((NPAD, V2), jnp.float32)),
        scratch_types=[
            pltpu.VMEM((b_per_w,), jnp.int32),
            pltpu.VMEM((b_per_w, V2), jnp.float32),
            pltpu.SemaphoreType.DMA,
        ],
    )
    def sc_gather(src_hbm, dst_hbm, sidx_hbm, didx_hbm,
                  out_src, out_dst, idx_v, rows, sem):
        wid = lax.axis_index("s") * nc + lax.axis_index("c")
        base = wid * b_per_w
        nactive = -(-n // b_per_w)

        @pl.when(wid < nactive)
        def _():
            for idx_hbm, table, out in ((sidx_hbm, src_hbm, out_src),
                                        (didx_hbm, dst_hbm, out_dst)):
                pltpu.sync_copy(idx_hbm.at[pl.ds(base, b_per_w)], idx_v)
                pltpu.async_copy(table.at[idx_v], rows, sem).wait()
                pltpu.sync_copy(rows, out.at[pl.ds(base, b_per_w)])

    return sc_gather


_GK = 0.7978845608028654
_GKA = _GK * 0.044715


def _tc_body(src_ref, dst_ref, pop_ref, dmat_ref,
             w1t_ref, w2h_ref, b2_ref, lab_ref, out_ref):
    pop = pop_ref[...]
    inv = jnp.where(pop == 0.0, 0.0,
                    1.0 / jnp.where(pop == 0.0, 1.0, pop))
    a = src_ref[...] * inv
    b = dst_ref[...] * inv

    acc = jnp.zeros((P, V2), jnp.float32)
    for h in range(HID):
        u = w1t_ref[h:h + 1, 0:1]
        w = w1t_ref[h:h + 1, 1:2]
        x = a * u + b * w + dmat_ref[h:h + 1, :]
        s = x * x
        th = jnp.tanh(x * (s * _GKA + _GK))
        v = x * w2h_ref[h:h + 1, 0:1]
        acc = acc + v + v * th
    logits = acc + b2_ref[0:1, 0:1]

    col = lax.broadcasted_iota(jnp.int32, (1, V2), 1)
    valid = col < NUM_ITEMS
    neg = jnp.where(valid, logits, -jnp.inf)
    m = jnp.max(neg, axis=1, keepdims=True)
    s = jnp.sum(jnp.where(valid, jnp.exp(logits - m), 0.0),
                axis=1, keepdims=True)
    lab = lab_ref[0]
    pick = jnp.sum(jnp.where(col == lab, logits, 0.0),
                   axis=1, keepdims=True)
    loss = jnp.log(s) + m - pick
    out_ref[...] = loss.reshape(1, P, 1)


def _tc_loss(src_g, dst_g, pop2, popn2, W1, b1, W2, b2, lab, n):
    nblk = n // P
    dmat = W1[2][:, None] * popn2 + b1[:, None]
    return pl.pallas_call(
        _tc_body,
        grid=(nblk,),
        in_specs=[
            pl.BlockSpec((P, V2), lambda i: (i, 0)),
            pl.BlockSpec((P, V2), lambda i: (i, 0)),
            pl.BlockSpec((1, V2), lambda i: (0, 0)),
            pl.BlockSpec((HID, V2), lambda i: (0, 0)),
            pl.BlockSpec((HID, 3), lambda i: (0, 0)),
            pl.BlockSpec((HID, 1), lambda i: (0, 0)),
            pl.BlockSpec((1, 1), lambda i: (0, 0)),
            pl.BlockSpec((1, P, 1), lambda i: (i, 0, 0)),
        ],
        out_specs=pl.BlockSpec((1, P, 1), lambda i: (i, 0, 0)),
        out_shape=jax.ShapeDtypeStruct((nblk, P, 1), jnp.float32),
    )(src_g, dst_g, pop2, dmat,
      W1.T, (0.5 * W2), b2.reshape(1, 1),
      lab.reshape(nblk, P, 1)).reshape(n)


@jax.jit
def kernel(st_src, st_dst, pop_biases, pop_biases_norm, W1, b1, W2, b2,
           masked_sequences, labels, positions):
    B, L = masked_sequences.shape
    n = B * L

    seqs = jnp.maximum(masked_sequences, 0)
    pad_col = jnp.full((B, 1), PAD_ID, dtype=seqs.dtype)
    src_idx = jnp.concatenate([pad_col, seqs[:, :-1]], axis=1).reshape(n)
    dst_idx = jnp.concatenate([seqs[:, 1:], pad_col], axis=1).reshape(n)
    zpad = jnp.zeros((NPAD - n,), jnp.int32)
    src_idx = jnp.concatenate([src_idx, zpad])
    dst_idx = jnp.concatenate([dst_idx, zpad])
    lab = jnp.maximum(labels, 0).reshape(n)

    cpad = ((0, 0), (0, V2 - V))
    src2 = jnp.pad(st_src, cpad)
    dst2 = jnp.pad(st_dst, cpad)
    pop2 = jnp.pad(pop_biases, cpad)
    popn2 = jnp.pad(pop_biases_norm.reshape(1, V), cpad)

    src_g, dst_g = _make_sc_gather(n)(src2, dst2, src_idx, dst_idx)

    return _tc_loss(src_g, dst_g, pop2, popn2, W1, b1, W2, b2, lab, n)
